# baseline (speedup 1.0000x reference)
import functools

import jax
import jax.numpy as jnp
from jax import lax
from jax.experimental import pallas as pl
from jax.experimental.pallas import tpu as pltpu
from jax.experimental.pallas import tpu_sc as plsc

N_NODES = 10000
DEG = 32
EDGE_DIM = 128
HID = EDGE_DIM // 2
KERNEL = 4
REDUCE = 8

BN = 500
NSPLIT = 5
NCH = N_NODES // NSPLIT
CHUNK = 64


def _mlp_kernel(x_ref, v1_ref, g1_ref, b1_ref, v2_ref, g2_ref,
                b2_ref, out_ref):
    v1 = v1_ref[...]
    g1 = g1_ref[...]
    v2 = v2_ref[...]
    g2 = g2_ref[...]

    n1 = jnp.sqrt(jnp.sum(v1 * v1, axis=1, keepdims=True))
    w1 = v1 * (g1 / (n1 + 1e-12))
    n2 = jnp.sqrt(jnp.sum(v2 * v2, axis=1, keepdims=True))
    w2 = v2 * (g2 / (n2 + 1e-12))

    x = x_ref[...]
    h = jax.lax.dot_general(x, w1, (((1,), (1,)), ((), ())),
                            preferred_element_type=jnp.float32)
    h = jax.nn.relu(h + b1_ref[...])
    logits = jax.lax.dot_general(h, w2, (((1,), (1,)), ((), ())),
                                 preferred_element_type=jnp.float32)
    out_ref[...] = logits + b2_ref[...]


def _sc_select(lg_hbm, out_hbm, slab, oslab):
    c = lax.axis_index("c")
    s_ = lax.axis_index("s")
    wid = s_ * 2 + c
    base = jnp.minimum(wid * CHUNK, NCH - CHUNK)
    pltpu.sync_copy(lg_hbm.at[pl.ds(base, CHUNK), :], slab)

    iota = lax.iota(jnp.int32, 16)
    rot8 = (iota + 8) & 15
    rot4 = (iota + 4) & 15
    rot2 = (iota + 2) & 15
    rot1 = (iota + 1) & 15
    xor1 = iota ^ 1
    xor2 = iota ^ 2
    rep4 = lax.shift_right_logical(iota, 2)
    mod4 = iota & 3

    def perm(x, idx):
        return lax.gather(
            x, idx[:, None],
            dimension_numbers=lax.GatherDimensionNumbers(
                offset_dims=(), collapsed_slice_dims=(0,),
                start_index_map=(0,)),
            slice_sizes=(1,),
            mode=lax.GatherScatterMode.PROMISE_IN_BOUNDS)

    def body(n, carry):
        v = [slab[n, pl.ds(16 * j, 16)] for j in range(8)]
        m = v[0]
        for j in range(1, 8):
            m = jnp.maximum(m, v[j])
        m = jnp.maximum(m, perm(m, rot8))
        m = jnp.maximum(m, perm(m, rot4))
        e = [jnp.exp(vj - m) for vj in v]
        s = e[0]
        for j in range(1, 8):
            s = s + e[j]
        s = s + perm(s, rot8)
        s = s + perm(s, rot4)
        ew = [ej / s for ej in e]
        sc = []
        for j in range(8):
            g = ew[j] + perm(ew[j], xor1)
            sc.append((g + perm(g, xor2)) * 0.25)
        sa = perm(sc[0], mod4 * 4)
        sb = perm(sc[4], mod4 * 4)
        for j in range(1, 4):
            sa = jnp.where(rep4 == j, perm(sc[j], mod4 * 4), sa)
            sb = jnp.where(rep4 == j, perm(sc[4 + j], mod4 * 4), sb)
        topd = iota
        for r in range(REDUCE):
            u = jnp.maximum(sa, sb)
            for rr in (rot8, rot4, rot2, rot1):
                u = jnp.maximum(u, perm(u, rr))
            cand = jnp.minimum(jnp.where(sa == u, iota, 99),
                               jnp.where(sb == u, iota + 16, 99))
            for rr in (rot8, rot4, rot2, rot1):
                cand = jnp.minimum(cand, perm(cand, rr))
            topd = jnp.where(iota == r, cand, topd)
            sa = jnp.where(iota == cand, -1.0, sa)
            sb = jnp.where(iota + 16 == cand, -1.0, sb)

        def gather_ew(drep):
            lane_idx = (drep & 3) * 4 + mod4
            jsel = lax.shift_right_logical(drep, 2)
            out = perm(ew[0], lane_idx)
            for j in range(1, 8):
                out = jnp.where(jsel == j, perm(ew[j], lane_idx), out)
            return out

        oslab[n, pl.ds(0, 16)] = gather_ew(perm(topd, rep4))
        oslab[n, pl.ds(16, 16)] = gather_ew(perm(topd, rep4 + 4))
        return carry

    lax.fori_loop(0, CHUNK, body, 0)
    pltpu.sync_copy(oslab, out_hbm.at[pl.ds(base, CHUNK), :])


_sc_select_call = functools.partial(
    pl.kernel,
    out_type=jax.ShapeDtypeStruct((NCH, REDUCE * KERNEL), jnp.float32),
    mesh=plsc.VectorSubcoreMesh(core_axis_name="c", subcore_axis_name="s"),
    scratch_types=[
        pltpu.VMEM((CHUNK, DEG * KERNEL), jnp.float32),
        pltpu.VMEM((CHUNK, REDUCE * KERNEL), jnp.float32),
    ],
)(_sc_select)


@jax.jit
def kernel(edge_feats, V1, g1, b1, V2, g2, b2):
    steps = NCH // BN
    parts = []
    for ci in range(NSPLIT):
        logits = pl.pallas_call(
            _mlp_kernel,
            grid=(steps,),
            in_specs=[
                pl.BlockSpec((BN * DEG, EDGE_DIM),
                             lambda i, ci=ci: (i + ci * steps, 0)),
                pl.BlockSpec((HID, EDGE_DIM), lambda i: (0, 0)),
                pl.BlockSpec((HID, 1), lambda i: (0, 0)),
                pl.BlockSpec((1, HID), lambda i: (0, 0)),
                pl.BlockSpec((KERNEL, HID), lambda i: (0, 0)),
                pl.BlockSpec((KERNEL, 1), lambda i: (0, 0)),
                pl.BlockSpec((1, KERNEL), lambda i: (0, 0)),
            ],
            out_specs=pl.BlockSpec((BN * DEG, KERNEL), lambda i: (i, 0)),
            out_shape=jax.ShapeDtypeStruct((NCH * DEG, KERNEL), jnp.float32),
        )(edge_feats, V1, g1.reshape(HID, 1), b1.reshape(1, HID),
          V2, g2.reshape(KERNEL, 1), b2.reshape(1, KERNEL))

        lg128 = logits.reshape(NCH, DEG * KERNEL)
        parts.append(_sc_select_call(lg128))
    out2d = jnp.concatenate(parts, axis=0)
    return out2d.reshape(N_NODES, REDUCE, KERNEL)


# speedup vs baseline: 1.0522x; 1.0522x over previous
import functools

import jax
import jax.numpy as jnp
from jax import lax
from jax.experimental import pallas as pl
from jax.experimental.pallas import tpu as pltpu
from jax.experimental.pallas import tpu_sc as plsc

N_NODES = 10000
DEG = 32
EDGE_DIM = 128
HID = EDGE_DIM // 2
KERNEL = 4
REDUCE = 8

BN = 500
CHUNKS = ((4000, 128), (4000, 128), (2000, 64))


def _mlp_kernel(x_ref, v1_ref, g1_ref, b1_ref, v2_ref, g2_ref,
                b2_ref, out_ref):
    v1 = v1_ref[...]
    g1 = g1_ref[...]
    v2 = v2_ref[...]
    g2 = g2_ref[...]

    n1 = jnp.sqrt(jnp.sum(v1 * v1, axis=1, keepdims=True))
    w1 = v1 * (g1 / (n1 + 1e-12))
    n2 = jnp.sqrt(jnp.sum(v2 * v2, axis=1, keepdims=True))
    w2 = v2 * (g2 / (n2 + 1e-12))

    x = x_ref[...]
    h = jax.lax.dot_general(x, w1, (((1,), (1,)), ((), ())),
                            preferred_element_type=jnp.float32)
    h = jax.nn.relu(h + b1_ref[...])
    logits = jax.lax.dot_general(h, w2, (((1,), (1,)), ((), ())),
                                 preferred_element_type=jnp.float32)
    out_ref[...] = logits + b2_ref[...]


def _sc_select_body(nch, chunk, lg_hbm, out_hbm, slab, oslab):
    c = lax.axis_index("c")
    s_ = lax.axis_index("s")
    wid = s_ * 2 + c
    base = jnp.minimum(wid * chunk, nch - chunk)
    pltpu.sync_copy(lg_hbm.at[pl.ds(base, chunk), :], slab)

    iota = lax.iota(jnp.int32, 16)
    rot8 = (iota + 8) & 15
    rot4 = (iota + 4) & 15
    rot2 = (iota + 2) & 15
    rot1 = (iota + 1) & 15
    xor1 = iota ^ 1
    xor2 = iota ^ 2
    rep4 = lax.shift_right_logical(iota, 2)
    mod4 = iota & 3

    def perm(x, idx):
        return lax.gather(
            x, idx[:, None],
            dimension_numbers=lax.GatherDimensionNumbers(
                offset_dims=(), collapsed_slice_dims=(0,),
                start_index_map=(0,)),
            slice_sizes=(1,),
            mode=lax.GatherScatterMode.PROMISE_IN_BOUNDS)

    def body(n, carry):
        v = [slab[n, pl.ds(16 * j, 16)] for j in range(8)]
        m = v[0]
        for j in range(1, 8):
            m = jnp.maximum(m, v[j])
        m = jnp.maximum(m, perm(m, rot8))
        m = jnp.maximum(m, perm(m, rot4))
        e = [jnp.exp(vj - m) for vj in v]
        s = e[0]
        for j in range(1, 8):
            s = s + e[j]
        s = s + perm(s, rot8)
        s = s + perm(s, rot4)
        ew = [ej / s for ej in e]
        sc = []
        for j in range(8):
            g = ew[j] + perm(ew[j], xor1)
            sc.append((g + perm(g, xor2)) * 0.25)
        sa = perm(sc[0], mod4 * 4)
        sb = perm(sc[4], mod4 * 4)
        for j in range(1, 4):
            sa = jnp.where(rep4 == j, perm(sc[j], mod4 * 4), sa)
            sb = jnp.where(rep4 == j, perm(sc[4 + j], mod4 * 4), sb)
        topd = iota
        for r in range(REDUCE):
            u = jnp.maximum(sa, sb)
            for rr in (rot8, rot4, rot2, rot1):
                u = jnp.maximum(u, perm(u, rr))
            cand = jnp.minimum(jnp.where(sa == u, iota, 99),
                               jnp.where(sb == u, iota + 16, 99))
            for rr in (rot8, rot4, rot2, rot1):
                cand = jnp.minimum(cand, perm(cand, rr))
            topd = jnp.where(iota == r, cand, topd)
            sa = jnp.where(iota == cand, -1.0, sa)
            sb = jnp.where(iota + 16 == cand, -1.0, sb)

        def gather_ew(drep):
            lane_idx = (drep & 3) * 4 + mod4
            jsel = lax.shift_right_logical(drep, 2)
            out = perm(ew[0], lane_idx)
            for j in range(1, 8):
                out = jnp.where(jsel == j, perm(ew[j], lane_idx), out)
            return out

        oslab[n, pl.ds(0, 16)] = gather_ew(perm(topd, rep4))
        oslab[n, pl.ds(16, 16)] = gather_ew(perm(topd, rep4 + 4))
        return carry

    lax.fori_loop(0, chunk, body, 0)
    pltpu.sync_copy(oslab, out_hbm.at[pl.ds(base, chunk), :])


@functools.cache
def _make_sc_select(nch, chunk):
    return functools.partial(
        pl.kernel,
        out_type=jax.ShapeDtypeStruct((nch, REDUCE * KERNEL), jnp.float32),
        mesh=plsc.VectorSubcoreMesh(core_axis_name="c",
                                    subcore_axis_name="s"),
        scratch_types=[
            pltpu.VMEM((chunk, DEG * KERNEL), jnp.float32),
            pltpu.VMEM((chunk, REDUCE * KERNEL), jnp.float32),
        ],
    )(functools.partial(_sc_select_body, nch, chunk))


@jax.jit
def kernel(edge_feats, V1, g1, b1, V2, g2, b2):
    parts = []
    row0 = 0
    for nch, chunk in CHUNKS:
        steps = nch // BN
        off = row0 // BN
        logits = pl.pallas_call(
            _mlp_kernel,
            grid=(steps,),
            in_specs=[
                pl.BlockSpec((BN * DEG, EDGE_DIM),
                             lambda i, off=off: (i + off, 0)),
                pl.BlockSpec((HID, EDGE_DIM), lambda i: (0, 0)),
                pl.BlockSpec((HID, 1), lambda i: (0, 0)),
                pl.BlockSpec((1, HID), lambda i: (0, 0)),
                pl.BlockSpec((KERNEL, HID), lambda i: (0, 0)),
                pl.BlockSpec((KERNEL, 1), lambda i: (0, 0)),
                pl.BlockSpec((1, KERNEL), lambda i: (0, 0)),
            ],
            out_specs=pl.BlockSpec((BN * DEG, KERNEL), lambda i: (i, 0)),
            out_shape=jax.ShapeDtypeStruct((nch * DEG, KERNEL), jnp.float32),
        )(edge_feats, V1, g1.reshape(HID, 1), b1.reshape(1, HID),
          V2, g2.reshape(KERNEL, 1), b2.reshape(1, KERNEL))

        lg128 = logits.reshape(nch, DEG * KERNEL)
        parts.append(_make_sc_select(nch, chunk)(lg128))
        row0 += nch
    out2d = jnp.concatenate(parts, axis=0)
    return out2d.reshape(N_NODES, REDUCE, KERNEL)


# speedup vs baseline: 1.0695x; 1.0165x over previous
import functools

import jax
import jax.numpy as jnp
from jax import lax
from jax.experimental import pallas as pl
from jax.experimental.pallas import tpu as pltpu
from jax.experimental.pallas import tpu_sc as plsc

N_NODES = 10000
DEG = 32
EDGE_DIM = 128
HID = EDGE_DIM // 2
KERNEL = 4
REDUCE = 8

BN = 500
CHUNKS = ((4000, 128), (5000, 160), (1000, 32))


def _mlp_kernel(x_ref, v1_ref, g1_ref, b1_ref, v2_ref, g2_ref,
                b2_ref, out_ref):
    v1 = v1_ref[...]
    g1 = g1_ref[...]
    v2 = v2_ref[...]
    g2 = g2_ref[...]

    n1 = jnp.sqrt(jnp.sum(v1 * v1, axis=1, keepdims=True))
    w1 = v1 * (g1 / (n1 + 1e-12))
    n2 = jnp.sqrt(jnp.sum(v2 * v2, axis=1, keepdims=True))
    w2 = v2 * (g2 / (n2 + 1e-12))

    x = x_ref[...]
    h = jax.lax.dot_general(x, w1, (((1,), (1,)), ((), ())),
                            preferred_element_type=jnp.float32)
    h = jax.nn.relu(h + b1_ref[...])
    logits = jax.lax.dot_general(h, w2, (((1,), (1,)), ((), ())),
                                 preferred_element_type=jnp.float32)
    out_ref[...] = logits + b2_ref[...]


def _sc_select_body(nch, chunk, lg_hbm, out_hbm, slab, oslab):
    c = lax.axis_index("c")
    s_ = lax.axis_index("s")
    wid = s_ * 2 + c
    base = jnp.minimum(wid * chunk, nch - chunk)
    pltpu.sync_copy(lg_hbm.at[pl.ds(base, chunk), :], slab)

    iota = lax.iota(jnp.int32, 16)
    rot8 = (iota + 8) & 15
    rot4 = (iota + 4) & 15
    rot2 = (iota + 2) & 15
    rot1 = (iota + 1) & 15
    xor1 = iota ^ 1
    xor2 = iota ^ 2
    rep4 = lax.shift_right_logical(iota, 2)
    mod4 = iota & 3

    def perm(x, idx):
        return lax.gather(
            x, idx[:, None],
            dimension_numbers=lax.GatherDimensionNumbers(
                offset_dims=(), collapsed_slice_dims=(0,),
                start_index_map=(0,)),
            slice_sizes=(1,),
            mode=lax.GatherScatterMode.PROMISE_IN_BOUNDS)

    def body(n, carry):
        v = [slab[n, pl.ds(16 * j, 16)] for j in range(8)]
        m = v[0]
        for j in range(1, 8):
            m = jnp.maximum(m, v[j])
        m = jnp.maximum(m, perm(m, rot8))
        m = jnp.maximum(m, perm(m, rot4))
        e = [jnp.exp(vj - m) for vj in v]
        s = e[0]
        for j in range(1, 8):
            s = s + e[j]
        s = s + perm(s, rot8)
        s = s + perm(s, rot4)
        ew = [ej / s for ej in e]
        sc = []
        for j in range(8):
            g = ew[j] + perm(ew[j], xor1)
            sc.append((g + perm(g, xor2)) * 0.25)
        sa = perm(sc[0], mod4 * 4)
        sb = perm(sc[4], mod4 * 4)
        for j in range(1, 4):
            sa = jnp.where(rep4 == j, perm(sc[j], mod4 * 4), sa)
            sb = jnp.where(rep4 == j, perm(sc[4 + j], mod4 * 4), sb)
        topd = iota
        for r in range(REDUCE):
            u = jnp.maximum(sa, sb)
            for rr in (rot8, rot4, rot2, rot1):
                u = jnp.maximum(u, perm(u, rr))
            cand = jnp.minimum(jnp.where(sa == u, iota, 99),
                               jnp.where(sb == u, iota + 16, 99))
            for rr in (rot8, rot4, rot2, rot1):
                cand = jnp.minimum(cand, perm(cand, rr))
            topd = jnp.where(iota == r, cand, topd)
            sa = jnp.where(iota == cand, -1.0, sa)
            sb = jnp.where(iota + 16 == cand, -1.0, sb)

        def gather_ew(drep):
            lane_idx = (drep & 3) * 4 + mod4
            jsel = lax.shift_right_logical(drep, 2)
            out = perm(ew[0], lane_idx)
            for j in range(1, 8):
                out = jnp.where(jsel == j, perm(ew[j], lane_idx), out)
            return out

        oslab[n, pl.ds(0, 16)] = gather_ew(perm(topd, rep4))
        oslab[n, pl.ds(16, 16)] = gather_ew(perm(topd, rep4 + 4))
        return carry

    lax.fori_loop(0, chunk, body, 0)
    pltpu.sync_copy(oslab, out_hbm.at[pl.ds(base, chunk), :])


@functools.cache
def _make_sc_select(nch, chunk):
    return functools.partial(
        pl.kernel,
        out_type=jax.ShapeDtypeStruct((nch, REDUCE * KERNEL), jnp.float32),
        mesh=plsc.VectorSubcoreMesh(core_axis_name="c",
                                    subcore_axis_name="s"),
        scratch_types=[
            pltpu.VMEM((chunk, DEG * KERNEL), jnp.float32),
            pltpu.VMEM((chunk, REDUCE * KERNEL), jnp.float32),
        ],
    )(functools.partial(_sc_select_body, nch, chunk))


@jax.jit
def kernel(edge_feats, V1, g1, b1, V2, g2, b2):
    parts = []
    row0 = 0
    for nch, chunk in CHUNKS:
        steps = nch // BN
        off = row0 // BN
        logits = pl.pallas_call(
            _mlp_kernel,
            grid=(steps,),
            in_specs=[
                pl.BlockSpec((BN * DEG, EDGE_DIM),
                             lambda i, off=off: (i + off, 0)),
                pl.BlockSpec((HID, EDGE_DIM), lambda i: (0, 0)),
                pl.BlockSpec((HID, 1), lambda i: (0, 0)),
                pl.BlockSpec((1, HID), lambda i: (0, 0)),
                pl.BlockSpec((KERNEL, HID), lambda i: (0, 0)),
                pl.BlockSpec((KERNEL, 1), lambda i: (0, 0)),
                pl.BlockSpec((1, KERNEL), lambda i: (0, 0)),
            ],
            out_specs=pl.BlockSpec((BN * DEG, KERNEL), lambda i: (i, 0)),
            out_shape=jax.ShapeDtypeStruct((nch * DEG, KERNEL), jnp.float32),
        )(edge_feats, V1, g1.reshape(HID, 1), b1.reshape(1, HID),
          V2, g2.reshape(KERNEL, 1), b2.reshape(1, KERNEL))

        lg128 = logits.reshape(nch, DEG * KERNEL)
        parts.append(_make_sc_select(nch, chunk)(lg128))
        row0 += nch
    out2d = jnp.concatenate(parts, axis=0)
    return out2d.reshape(N_NODES, REDUCE, KERNEL)
